# baseline (speedup 1.0000x reference)
import functools

import jax
import jax.numpy as jnp
from jax import lax
from jax.experimental import pallas as pl
from jax.experimental.pallas import tpu as pltpu
from jax.experimental.pallas import tpu_sc as plsc

_BATCH = 16384
_N_FIELDS = 26
_OUT_DIM = 32
_TOTAL = _BATCH * _N_FIELDS
_INPUT_DIM = 1000000

_NC = 2
_NS = 16
_NW = _NC * _NS
_PER_W = _TOTAL // _NW
_C = 832
_K = _PER_W // _C
_H = 4
_G = _K // _H

_BLK = 8192

assert _PER_W * _NW == _TOTAL
assert _K * _C == _PER_W
assert _G * _H == _K


def _conv_body(x_ref, o_ref):
    x = x_ref[...]
    xt = jnp.swapaxes(x, 0, 1)
    z = xt.reshape(_BLK // 4, 4, _OUT_DIM)
    o_ref[...] = jnp.concatenate([z[:, 0], z[:, 1], z[:, 2], z[:, 3]], axis=1)


def _relayout_table(table):
    table_t = table.T
    grid = pl.cdiv(_INPUT_DIM, _BLK)
    conv = pl.pallas_call(
        _conv_body,
        grid=(grid,),
        in_specs=[pl.BlockSpec((_OUT_DIM, _BLK), lambda i: (0, i))],
        out_specs=pl.BlockSpec((_BLK // 4, 128), lambda i: (i, 0)),
        out_shape=jax.ShapeDtypeStruct((_INPUT_DIM // 4, 128), jnp.float32),
    )(table_t)
    return conv.reshape(_INPUT_DIM, _OUT_DIM)


_CB = 1024
_NQ = _TOTAL // _CB
_KW = _NQ // _NW


def _sc_gather(idxT, table_rm):
    mesh = plsc.VectorSubcoreMesh(core_axis_name="c", subcore_axis_name="s")

    @functools.partial(
        pl.kernel,
        out_type=jax.ShapeDtypeStruct((_N_FIELDS, _OUT_DIM, _BATCH), jnp.float32),
        mesh=mesh,
        compiler_params=pltpu.CompilerParams(
            use_tc_tiling_on_sc=False, needs_layout_passes=False
        ),
        scratch_types=[
            pltpu.VMEM((_KW, _CB), jnp.int32),
            pltpu.VMEM((2, _CB, _OUT_DIM), jnp.float32),
            pltpu.VMEM((1, _OUT_DIM, _CB), jnp.float32),
            pltpu.SemaphoreType.DMA,
            pltpu.SemaphoreType.DMA,
        ],
    )
    def body(idx_hbm, table_hbm, out_hbm, idx_v, rows_v, tbuf, sem0, sem1):
        sems = (sem0, sem1)
        wid = lax.axis_index("s") * _NC + lax.axis_index("c")
        pltpu.sync_copy(idx_hbm.at[pl.ds(wid * _KW, _KW)], idx_v)
        q0 = wid * _KW
        iota16 = lax.iota(jnp.int32, 16)

        def fire(j):
            pltpu.async_copy(table_hbm.at[idx_v.at[j]], rows_v.at[j % 2], sems[j % 2])

        def wait(j):
            pltpu.make_async_copy(
                table_hbm.at[idx_v.at[0]], rows_v.at[j % 2], sems[j % 2]
            ).wait()

        fire(0)
        for j in range(_KW):
            wait(j)
            if j + 1 < _KW:
                fire(j + 1)
            src = rows_v.at[j % 2]

            def tstep(t, carry):
                c = t >> 6
                b0 = (t & 63) << 4
                bidx = iota16 + b0
                cidx = iota16 * 0 + c
                v = plsc.load_gather(src, [bidx, cidx])
                tbuf[0, c, pl.ds(b0, 16)] = v
                return carry

            lax.fori_loop(0, (_OUT_DIM * _CB) // 16, tstep, 0, unroll=16)

            q = q0 + j
            f = q >> 4
            b_start = (q & 15) * _CB
            pltpu.sync_copy(
                tbuf, out_hbm.at[pl.ds(f, 1), :, pl.ds(b_start, _CB)]
            )

    return body(idxT, table_rm)


@jax.jit
def _impl(indices, embedding_table):
    idxT = jnp.transpose(indices.astype(jnp.int32)).reshape(_NQ, _CB)
    table_rm = _relayout_table(embedding_table)
    out3 = _sc_gather(idxT, table_rm)
    return jnp.transpose(out3, (2, 0, 1))


def kernel(indices, embedding_table):
    return _impl(indices, embedding_table)


# speedup vs baseline: 1.0795x; 1.0795x over previous
import functools

import jax
import jax.numpy as jnp
from jax import lax
from jax.experimental import pallas as pl
from jax.experimental.pallas import tpu as pltpu
from jax.experimental.pallas import tpu_sc as plsc

_BATCH = 16384
_N_FIELDS = 26
_OUT_DIM = 32
_TOTAL = _BATCH * _N_FIELDS
_INPUT_DIM = 1000000

_NC = 2
_NS = 16
_NW = _NC * _NS
_PER_W = _TOTAL // _NW
_C = 832
_K = _PER_W // _C
_H = 4
_G = _K // _H

_BLK = 8192

assert _PER_W * _NW == _TOTAL
assert _K * _C == _PER_W
assert _G * _H == _K


def _conv_body(x_ref, o_ref):
    x = x_ref[...]
    xt = jnp.swapaxes(x, 0, 1)
    z = xt.reshape(_BLK // 4, 4, _OUT_DIM)
    o_ref[...] = jnp.concatenate([z[:, 0], z[:, 1], z[:, 2], z[:, 3]], axis=1)


def _relayout_table(table):
    table_t = table.T
    grid = pl.cdiv(_INPUT_DIM, _BLK)
    conv = pl.pallas_call(
        _conv_body,
        grid=(grid,),
        in_specs=[pl.BlockSpec((_OUT_DIM, _BLK), lambda i: (0, i))],
        out_specs=pl.BlockSpec((_BLK // 4, 128), lambda i: (i, 0)),
        out_shape=jax.ShapeDtypeStruct((_INPUT_DIM // 4, 128), jnp.float32),
    )(table_t)
    return conv.reshape(_INPUT_DIM, _OUT_DIM)


_CB = 1024
_NQ = _TOTAL // _CB
_KW = _NQ // _NW


def _sc_gather(idxT, table_rm):
    mesh = plsc.VectorSubcoreMesh(core_axis_name="c", subcore_axis_name="s")

    @functools.partial(
        pl.kernel,
        out_type=jax.ShapeDtypeStruct((_N_FIELDS, _OUT_DIM, _BATCH), jnp.float32),
        mesh=mesh,
        compiler_params=pltpu.CompilerParams(
            use_tc_tiling_on_sc=False, needs_layout_passes=False
        ),
        scratch_types=[
            pltpu.VMEM((_KW, _CB), jnp.int32),
            pltpu.VMEM((2, _CB, _OUT_DIM), jnp.float32),
            pltpu.VMEM((_OUT_DIM * _CB,), jnp.float32),
            pltpu.SemaphoreType.DMA,
            pltpu.SemaphoreType.DMA,
            pltpu.SemaphoreType.DMA,
        ],
    )
    def body(idx_hbm, table_hbm, out_hbm, idx_v, rows_v, tbuf, sem0, sem1, sem_out):
        sems = (sem0, sem1)
        wid = lax.axis_index("s") * _NC + lax.axis_index("c")
        pltpu.sync_copy(idx_hbm.at[pl.ds(wid * _KW, _KW)], idx_v)
        q0 = wid * _KW
        iota16 = lax.iota(jnp.int32, 16)
        colvec = iota16 * _CB

        def fire(j):
            pltpu.async_copy(table_hbm.at[idx_v.at[j]], rows_v.at[j % 2], sems[j % 2])

        def wait(j):
            pltpu.make_async_copy(
                table_hbm.at[idx_v.at[0]], rows_v.at[j % 2], sems[j % 2]
            ).wait()

        fire(0)
        for j in range(_KW):
            wait(j)
            if j + 1 < _KW:
                fire(j + 1)
            jm = j % 2

            def tstep(t, carry):
                b = t >> 1
                c0 = (t & 1) << 4
                v = rows_v[jm, b, pl.ds(c0, 16)]
                plsc.store_scatter(tbuf, [colvec + (c0 * _CB + b)], v)
                return carry

            lax.fori_loop(0, 2 * _CB, tstep, 0, unroll=8)

            q = q0 + j
            f = q >> 4
            b_start = (q & 15) * _CB
            for c in range(_OUT_DIM):
                pltpu.async_copy(
                    tbuf.at[pl.ds(c * _CB, _CB)],
                    out_hbm.at[f, c, pl.ds(b_start, _CB)],
                    sem_out,
                )
            for c in range(_OUT_DIM):
                pltpu.make_async_copy(
                    tbuf.at[pl.ds(0, _CB)],
                    out_hbm.at[f, 0, pl.ds(b_start, _CB)],
                    sem_out,
                ).wait()

    return body(idxT, table_rm)


@jax.jit
def _impl(indices, embedding_table):
    idxT = jnp.transpose(indices.astype(jnp.int32)).reshape(_NQ, _CB)
    table_rm = _relayout_table(embedding_table)
    out3 = _sc_gather(idxT, table_rm)
    return jnp.transpose(out3, (2, 0, 1))


def kernel(indices, embedding_table):
    return _impl(indices, embedding_table)


# speedup vs baseline: 1.3458x; 1.2467x over previous
import functools

import jax
import jax.numpy as jnp
from jax import lax
from jax.experimental import pallas as pl
from jax.experimental.pallas import tpu as pltpu
from jax.experimental.pallas import tpu_sc as plsc

_BATCH = 16384
_N_FIELDS = 26
_OUT_DIM = 32
_TOTAL = _BATCH * _N_FIELDS
_INPUT_DIM = 1000000

_NC = 2
_NS = 16
_NW = _NC * _NS
_PER_W = _TOTAL // _NW
_C = 832
_K = _PER_W // _C
_H = 4
_G = _K // _H

_BLK = 8192

assert _PER_W * _NW == _TOTAL
assert _K * _C == _PER_W
assert _G * _H == _K


def _conv_body(x_ref, o_ref):
    x = x_ref[...]
    xt = jnp.swapaxes(x, 0, 1)
    z = xt.reshape(_BLK // 4, 4, _OUT_DIM)
    o_ref[...] = jnp.concatenate([z[:, 0], z[:, 1], z[:, 2], z[:, 3]], axis=1)


def _relayout_table(table):
    table_t = table.T
    grid = pl.cdiv(_INPUT_DIM, _BLK)
    conv = pl.pallas_call(
        _conv_body,
        grid=(grid,),
        in_specs=[pl.BlockSpec((_OUT_DIM, _BLK), lambda i: (0, i))],
        out_specs=pl.BlockSpec((_BLK // 4, 128), lambda i: (i, 0)),
        out_shape=jax.ShapeDtypeStruct((_INPUT_DIM // 4, 128), jnp.float32),
    )(table_t)
    return conv.reshape(_INPUT_DIM, _OUT_DIM)


_CB = 1024
_NQ = _TOTAL // _CB
_KW = _NQ // _NW


def _sc_gather(idxT, table_rm):
    mesh = plsc.VectorSubcoreMesh(core_axis_name="c", subcore_axis_name="s")

    @functools.partial(
        pl.kernel,
        out_type=jax.ShapeDtypeStruct((_N_FIELDS, _OUT_DIM, _BATCH), jnp.float32),
        mesh=mesh,
        compiler_params=pltpu.CompilerParams(
            use_tc_tiling_on_sc=False, needs_layout_passes=False
        ),
        scratch_types=[
            pltpu.VMEM((_KW, _CB), jnp.int32),
            pltpu.VMEM((2, _CB, _OUT_DIM), jnp.float32),
            pltpu.VMEM((_OUT_DIM * _CB,), jnp.float32),
            pltpu.SemaphoreType.DMA,
            pltpu.SemaphoreType.DMA,
            pltpu.SemaphoreType.DMA,
        ],
    )
    def body(idx_hbm, table_hbm, out_hbm, idx_v, rows_v, tbuf, sem0, sem1, sem_out):
        sems = (sem0, sem1)
        wid = lax.axis_index("s") * _NC + lax.axis_index("c")
        pltpu.sync_copy(idx_hbm.at[pl.ds(wid * _KW, _KW)], idx_v)
        q0 = wid * _KW
        iota16 = lax.iota(jnp.int32, 16)
        src_vecs = []
        dst_vecs = []
        for k in range(16):
            perm_k = (iota16 + k) & 15
            src_vecs.append(perm_k)
            dst_vecs.append(perm_k * _CB + iota16)

        def fire(j):
            pltpu.async_copy(table_hbm.at[idx_v.at[j]], rows_v.at[j % 2], sems[j % 2])

        def wait(j):
            pltpu.make_async_copy(
                table_hbm.at[idx_v.at[0]], rows_v.at[j % 2], sems[j % 2]
            ).wait()

        fire(0)
        for j in range(_KW):
            wait(j)
            if j + 1 < _KW:
                fire(j + 1)
            src = rows_v.at[j % 2]

            def tstep(t, carry):
                b0 = (t >> 1) << 4
                c0 = (t & 1) << 4
                bidx = iota16 + b0
                doff = c0 * _CB + b0
                for k in range(16):
                    v = plsc.load_gather(src, [bidx, src_vecs[k] + c0])
                    plsc.store_scatter(tbuf, [dst_vecs[k] + doff], v)
                return carry

            lax.fori_loop(0, (2 * _CB) // 16, tstep, 0, unroll=2)

            q = q0 + j
            f = q >> 4
            b_start = (q & 15) * _CB
            for c in range(_OUT_DIM):
                pltpu.async_copy(
                    tbuf.at[pl.ds(c * _CB, _CB)],
                    out_hbm.at[f, c, pl.ds(b_start, _CB)],
                    sem_out,
                )
            for c in range(_OUT_DIM):
                pltpu.make_async_copy(
                    tbuf.at[pl.ds(0, _CB)],
                    out_hbm.at[f, 0, pl.ds(b_start, _CB)],
                    sem_out,
                ).wait()

    return body(idxT, table_rm)


@jax.jit
def _impl(indices, embedding_table):
    idxT = jnp.transpose(indices.astype(jnp.int32)).reshape(_NQ, _CB)
    table_rm = _relayout_table(embedding_table)
    out3 = _sc_gather(idxT, table_rm)
    return jnp.transpose(out3, (2, 0, 1))


def kernel(indices, embedding_table):
    return _impl(indices, embedding_table)


# speedup vs baseline: 1.5810x; 1.1747x over previous
import functools

import jax
import jax.numpy as jnp
from jax import lax
from jax.experimental import pallas as pl
from jax.experimental.pallas import tpu as pltpu
from jax.experimental.pallas import tpu_sc as plsc

_BATCH = 16384
_N_FIELDS = 26
_OUT_DIM = 32
_TOTAL = _BATCH * _N_FIELDS
_INPUT_DIM = 1000000

_NC = 2
_NS = 16
_NW = _NC * _NS
_PER_W = _TOTAL // _NW
_C = 832
_K = _PER_W // _C
_H = 4
_G = _K // _H

_BLK = 8192

assert _PER_W * _NW == _TOTAL
assert _K * _C == _PER_W
assert _G * _H == _K


_NJ = 7813


def _relayout_table(table):
    table_t = table.T
    mesh = plsc.VectorSubcoreMesh(core_axis_name="c", subcore_axis_name="s")

    @functools.partial(
        pl.kernel,
        out_type=jax.ShapeDtypeStruct((_INPUT_DIM // 4, 128), jnp.float32),
        mesh=mesh,
        compiler_params=pltpu.CompilerParams(
            use_tc_tiling_on_sc=True, needs_layout_passes=False
        ),
        scratch_types=[
            pltpu.VMEM((2, _OUT_DIM, 128), jnp.float32),
            pltpu.VMEM((2, _OUT_DIM, 128), jnp.float32),
            pltpu.SemaphoreType.DMA,
            pltpu.SemaphoreType.DMA,
        ],
    )
    def body(tt_hbm, out_hbm, vbuf, obuf, sem0, sem1):
        sems = (sem0, sem1)
        wid = lax.axis_index("s") * _NC + lax.axis_index("c")
        iota16 = lax.iota(jnp.int32, 16)
        perms = []
        mvecs = []
        for k in range(16):
            perm_k = (iota16 + k) & 15
            perms.append(perm_k)
            mvecs.append((iota16 & 3) * _OUT_DIM + perm_k)
        qpat = (iota16 >> 2)

        def tile_of(t):
            return wid + _NW * t

        def fire_in(t, slot):
            j = tile_of(t)

            @pl.when(j < _NJ)
            def _():
                pltpu.async_copy(
                    tt_hbm.at[:, pl.ds(j * 128, 128)], vbuf.at[slot], sems[slot]
                )

        def wait_in(t, slot):
            j = tile_of(t)

            @pl.when(j < _NJ)
            def _():
                pltpu.make_async_copy(
                    tt_hbm.at[:, pl.ds(0, 128)], vbuf.at[slot], sems[slot]
                ).wait()

        def work(t, slot):
            j = tile_of(t)
            vb = vbuf.at[slot]
            ob = obuf.at[slot]

            def blk(tb, carry):
                c0 = (tb & 1) << 4
                r0 = (tb >> 1) << 4
                ridx = iota16 + r0
                qvec = qpat + (r0 >> 2)
                for k in range(16):
                    v = plsc.load_gather(vb, [perms[k] + c0, ridx])
                    plsc.store_scatter(ob, [qvec, mvecs[k] + c0], v)
                return carry

            @pl.when(j < _NJ)
            def _():
                lax.fori_loop(0, 16, blk, 0, unroll=2)

            @pl.when(j < _NJ - 1)
            def _():
                pltpu.sync_copy(ob, out_hbm.at[pl.ds(j * 32, 32)])

            @pl.when(j == _NJ - 1)
            def _():
                pltpu.sync_copy(
                    ob.at[pl.ds(0, 16)], out_hbm.at[pl.ds(j * 32, 16)]
                )

        fire_in(0, 0)

        def step(i, carry):
            t0 = 2 * i
            fire_in(t0 + 1, 1)
            wait_in(t0, 0)
            work(t0, 0)
            fire_in(t0 + 2, 0)
            wait_in(t0 + 1, 1)
            work(t0 + 1, 1)
            return carry

        lax.fori_loop(0, 123, step, 0)

    conv = body(table_t)
    return conv.reshape(_INPUT_DIM, _OUT_DIM)


_CB = 1024
_NQ = _TOTAL // _CB
_KW = _NQ // _NW


def _sc_gather(idxT, table_rm):
    mesh = plsc.VectorSubcoreMesh(core_axis_name="c", subcore_axis_name="s")

    @functools.partial(
        pl.kernel,
        out_type=jax.ShapeDtypeStruct((_N_FIELDS, _OUT_DIM, _BATCH), jnp.float32),
        mesh=mesh,
        compiler_params=pltpu.CompilerParams(
            use_tc_tiling_on_sc=False, needs_layout_passes=False
        ),
        scratch_types=[
            pltpu.VMEM((_KW, _CB), jnp.int32),
            pltpu.VMEM((2, _CB, _OUT_DIM), jnp.float32),
            pltpu.VMEM((_OUT_DIM * _CB,), jnp.float32),
            pltpu.SemaphoreType.DMA,
            pltpu.SemaphoreType.DMA,
            pltpu.SemaphoreType.DMA,
        ],
    )
    def body(idx_hbm, table_hbm, out_hbm, idx_v, rows_v, tbuf, sem0, sem1, sem_out):
        sems = (sem0, sem1)
        wid = lax.axis_index("s") * _NC + lax.axis_index("c")
        pltpu.sync_copy(idx_hbm.at[pl.ds(wid * _KW, _KW)], idx_v)
        q0 = wid * _KW
        iota16 = lax.iota(jnp.int32, 16)
        src_vecs = []
        dst_vecs = []
        for k in range(16):
            perm_k = (iota16 + k) & 15
            src_vecs.append(perm_k)
            dst_vecs.append(perm_k * _CB + iota16)

        def fire(j):
            pltpu.async_copy(table_hbm.at[idx_v.at[j]], rows_v.at[j % 2], sems[j % 2])

        def wait(j):
            pltpu.make_async_copy(
                table_hbm.at[idx_v.at[0]], rows_v.at[j % 2], sems[j % 2]
            ).wait()

        fire(0)
        for j in range(_KW):
            wait(j)
            if j + 1 < _KW:
                fire(j + 1)
            src = rows_v.at[j % 2]

            def tstep(t, carry):
                b0 = (t >> 1) << 4
                c0 = (t & 1) << 4
                bidx = iota16 + b0
                doff = c0 * _CB + b0
                for k in range(16):
                    v = plsc.load_gather(src, [bidx, src_vecs[k] + c0])
                    plsc.store_scatter(tbuf, [dst_vecs[k] + doff], v)
                return carry

            lax.fori_loop(0, (2 * _CB) // 16, tstep, 0, unroll=2)

            q = q0 + j
            f = q >> 4
            b_start = (q & 15) * _CB
            for c in range(_OUT_DIM):
                pltpu.async_copy(
                    tbuf.at[pl.ds(c * _CB, _CB)],
                    out_hbm.at[f, c, pl.ds(b_start, _CB)],
                    sem_out,
                )
            for c in range(_OUT_DIM):
                pltpu.make_async_copy(
                    tbuf.at[pl.ds(0, _CB)],
                    out_hbm.at[f, 0, pl.ds(b_start, _CB)],
                    sem_out,
                ).wait()

    return body(idxT, table_rm)


@jax.jit
def _impl(indices, embedding_table):
    idxT = jnp.transpose(indices.astype(jnp.int32)).reshape(_NQ, _CB)
    table_rm = _relayout_table(embedding_table)
    out3 = _sc_gather(idxT, table_rm)
    return jnp.transpose(out3, (2, 0, 1))


def kernel(indices, embedding_table):
    return _impl(indices, embedding_table)
